# Initial kernel scaffold; baseline (speedup 1.0000x reference)
import jax
import jax.numpy as jnp
from jax import lax
from jax.experimental import pallas as pl
from jax.experimental.pallas import tpu as pltpu
from jax.experimental.pallas import tpu_sc as plsc

B = 4096
S = 200
D = 64
N = B * S
NC = 2
NS = 16
NW = NC * NS
NPW = N // NW
K = 800
SEQ_PER_CHUNK = K // S
CHUNKS = NPW // K
LANES = 16


def _body(idx_hbm, table_hbm, pos_hbm, out_hbm, pos_v, idx_v, rows_v, gsem):
    wid = lax.axis_index("s") * NC + lax.axis_index("c")
    base = wid * NPW
    pltpu.sync_copy(pos_hbm, pos_v)

    @pl.loop(0, CHUNKS)
    def _chunk(g):
        cbase = base + g * K
        pltpu.sync_copy(idx_hbm.at[pl.ds(cbase, K)], idx_v)
        pltpu.async_copy(table_hbm.at[idx_v], rows_v, gsem).wait()

        @pl.loop(0, S)
        def _row(p):
            for c in range(D // LANES):
                pv = pos_v[p, pl.ds(c * LANES, LANES)]
                for sq in range(SEQ_PER_CHUNK):
                    r = sq * S + p
                    rows_v[r, pl.ds(c * LANES, LANES)] += pv

        pltpu.sync_copy(rows_v, out_hbm.at[pl.ds(cbase, K)])


@jax.jit
def kernel(x, table, pos_encoding):
    idx = x.reshape(-1).astype(jnp.int32)
    pos = pos_encoding[:S].astype(jnp.float32)
    mesh = plsc.VectorSubcoreMesh(core_axis_name="c", subcore_axis_name="s")
    out = pl.kernel(
        _body,
        out_type=jax.ShapeDtypeStruct((N, D), jnp.float32),
        mesh=mesh,
        scratch_types=[
            pltpu.VMEM((S, D), jnp.float32),
            pltpu.VMEM((K,), jnp.int32),
            pltpu.VMEM((K, D), jnp.float32),
            pltpu.SemaphoreType.DMA,
        ],
    )(idx, table, pos)
    return out.reshape(B, S, D)


# speedup vs baseline: 3.7000x; 3.7000x over previous
import jax
import jax.numpy as jnp
from jax import lax
from jax.experimental import pallas as pl
from jax.experimental.pallas import tpu as pltpu
from jax.experimental.pallas import tpu_sc as plsc

B = 4096
S = 200
D = 64
N = B * S
NC = 2
NS = 16
NW = NC * NS
NPW = N // NW
K = 800
SEQ_PER_CHUNK = K // S
CHUNKS = NPW // K
LANES = 16


def _body(idx_hbm, table_hbm, pos_hbm, out_hbm, pos_v, idx_v, rows_v, gsem):
    wid = lax.axis_index("s") * NC + lax.axis_index("c")
    base = wid * NPW
    pltpu.sync_copy(pos_hbm, pos_v)

    @pl.loop(0, CHUNKS)
    def _chunk(g):
        cbase = base + g * K
        pltpu.sync_copy(idx_hbm.at[pl.ds(cbase, K)], idx_v)
        pltpu.async_copy(table_hbm.at[idx_v], rows_v, gsem).wait()

        @pl.loop(0, S)
        def _row(p):
            for c in range(D // LANES):
                pv = pos_v[p, pl.ds(c * LANES, LANES)]
                for sq in range(SEQ_PER_CHUNK):
                    r = sq * S + p
                    rows_v[r, pl.ds(c * LANES, LANES)] += pv

        pltpu.sync_copy(rows_v, out_hbm.at[pl.ds(cbase, K)])


@jax.jit
def kernel(x, table, pos_encoding):
    idx = x.reshape(-1).astype(jnp.int32)
    pos = pos_encoding[:S].astype(jnp.float32)
    mesh = plsc.VectorSubcoreMesh(core_axis_name="c", subcore_axis_name="s")
    out = pl.kernel(
        _body,
        out_type=jax.ShapeDtypeStruct((N, D), jnp.float32),
        mesh=mesh,
        compiler_params=pltpu.CompilerParams(use_tc_tiling_on_sc=False),
        scratch_types=[
            pltpu.VMEM((S, D), jnp.float32),
            pltpu.VMEM((K,), jnp.int32),
            pltpu.VMEM((K, D), jnp.float32),
            pltpu.SemaphoreType.DMA,
        ],
    )(idx, table, pos)
    return out.reshape(B, S, D)


# speedup vs baseline: 3.9999x; 1.0811x over previous
import jax
import jax.numpy as jnp
from jax import lax
from jax.experimental import pallas as pl
from jax.experimental.pallas import tpu as pltpu
from jax.experimental.pallas import tpu_sc as plsc

B = 4096
S = 200
D = 64
N = B * S
NC = 2
NS = 16
NW = NC * NS
NPW = N // NW
K = 200
NBUF = 4
CHUNKS = NPW // K
LANES = 16


def _body(idx_hbm, table_hbm, pos_hbm, out_hbm, pos_v, idx_all, *bufs):
    rows = bufs[0:NBUF]
    gsems = bufs[NBUF:2 * NBUF]
    wsems = bufs[2 * NBUF:3 * NBUF]

    wid = lax.axis_index("s") * NC + lax.axis_index("c")
    base = wid * NPW
    pltpu.sync_copy(pos_hbm, pos_v)
    pltpu.sync_copy(idx_hbm.at[pl.ds(base, NPW)], idx_all)

    def wait_bytes(sem, dst):
        pltpu.make_async_copy(out_hbm.at[pl.ds(base, K)], dst, sem).wait()

    def wait_write(b):
        pltpu.make_async_copy(rows[b], out_hbm.at[pl.ds(base, K)], wsems[b]).wait()

    for b in range(NBUF):
        pltpu.async_copy(table_hbm.at[idx_all.at[pl.ds(b * K, K)]],
                         rows[b], gsems[b])

    @pl.loop(0, CHUNKS, step=NBUF)
    def _grp(g0):
        for b in range(NBUF):
            g = g0 + b
            cbase = base + g * K
            wait_bytes(gsems[b], rows[b])

            @pl.loop(0, S)
            def _row(p):
                for c in range(D // LANES):
                    sl = pl.ds(c * LANES, LANES)
                    rows[b][p, sl] += pos_v[p, sl]

            pltpu.async_copy(rows[b], out_hbm.at[pl.ds(cbase, K)], wsems[b])

            @pl.when(g + NBUF < CHUNKS)
            def _():
                wait_write(b)
                pltpu.async_copy(
                    table_hbm.at[idx_all.at[pl.ds((g + NBUF) * K, K)]],
                    rows[b], gsems[b])

    for b in range(NBUF):
        wait_write(b)


@jax.jit
def kernel(x, table, pos_encoding):
    idx = x.reshape(-1).astype(jnp.int32)
    pos = pos_encoding[:S].astype(jnp.float32)
    mesh = plsc.VectorSubcoreMesh(core_axis_name="c", subcore_axis_name="s")
    out = pl.kernel(
        _body,
        out_type=jax.ShapeDtypeStruct((N, D), jnp.float32),
        mesh=mesh,
        compiler_params=pltpu.CompilerParams(use_tc_tiling_on_sc=False),
        scratch_types=[
            pltpu.VMEM((S, D), jnp.float32),
            pltpu.VMEM((NPW,), jnp.int32),
        ] + [pltpu.VMEM((K, D), jnp.float32) for _ in range(NBUF)]
          + [pltpu.SemaphoreType.DMA for _ in range(2 * NBUF)],
    )(idx, table, pos)
    return out.reshape(B, S, D)
